# baseline (speedup 1.0000x reference)
import jax
import jax.numpy as jnp
from jax.experimental import pallas as pl

_ROWS = 16384
_COLS = 1000
_BLOCK_ROWS = 256


def _body(sav_ref, act_ref, qp_ref, out_ref):
    cols = jax.lax.broadcasted_iota(jnp.int32, sav_ref.shape, 1)
    out_ref[...] = jnp.where(cols == act_ref[...], qp_ref[...], sav_ref[...])


def kernel(state_action_values, action, q_prime):
    qp2 = q_prime.reshape(_ROWS, 1)
    grid = (_ROWS // _BLOCK_ROWS,)
    return pl.pallas_call(
        _body,
        grid=grid,
        in_specs=[
            pl.BlockSpec((_BLOCK_ROWS, _COLS), lambda i: (i, 0)),
            pl.BlockSpec((_BLOCK_ROWS, 1), lambda i: (i, 0)),
            pl.BlockSpec((_BLOCK_ROWS, 1), lambda i: (i, 0)),
        ],
        out_specs=pl.BlockSpec((_BLOCK_ROWS, _COLS), lambda i: (i, 0)),
        out_shape=jax.ShapeDtypeStruct((_ROWS, _COLS), jnp.float32),
    )(state_action_values, action, qp2)


# speedup vs baseline: 1.1251x; 1.1251x over previous
import jax
import jax.numpy as jnp
from jax.experimental import pallas as pl

_ROWS = 16384
_COLS = 1000
_BLOCK_ROWS = 2048


def _body(sav_ref, act_ref, qp_ref, out_ref):
    cols = jax.lax.broadcasted_iota(jnp.int32, sav_ref.shape, 1)
    out_ref[...] = jnp.where(cols == act_ref[...], qp_ref[...], sav_ref[...])


def kernel(state_action_values, action, q_prime):
    qp2 = q_prime.reshape(_ROWS, 1)
    grid = (_ROWS // _BLOCK_ROWS,)
    return pl.pallas_call(
        _body,
        grid=grid,
        in_specs=[
            pl.BlockSpec((_BLOCK_ROWS, _COLS), lambda i: (i, 0)),
            pl.BlockSpec((_BLOCK_ROWS, 1), lambda i: (i, 0)),
            pl.BlockSpec((_BLOCK_ROWS, 1), lambda i: (i, 0)),
        ],
        out_specs=pl.BlockSpec((_BLOCK_ROWS, _COLS), lambda i: (i, 0)),
        out_shape=jax.ShapeDtypeStruct((_ROWS, _COLS), jnp.float32),
    )(state_action_values, action, qp2)
